# baseline (speedup 1.0000x reference)
import functools
import math

import jax
import jax.numpy as jnp
from jax import lax
from jax.experimental import pallas as pl
from jax.experimental.pallas import tpu as pltpu
from jax.experimental.pallas import tpu_sc as plsc

B = 2
N = 16384
D = 512
H = 8
P = 8
DH = 64
HS = 128
LQ = 2 * H * P


MA = 256


def _geom_body(q_ref, ref_ref, woff_ref, boff_ref, wattn2_ref, battn2_ref,
               p0_ref, p1_ref):
    q = q_ref[...]
    lo = jnp.dot(q, woff_ref[...], preferred_element_type=jnp.float32) + boff_ref[...]
    la = jnp.dot(q, wattn2_ref[...], preferred_element_type=jnp.float32) + battn2_ref[...]
    m = jnp.max(la, axis=-1, keepdims=True)
    e = jnp.exp(la - m)
    r128 = lax.broadcasted_iota(jnp.int32, (LQ, LQ), 0) // 16
    c128 = lax.broadcasted_iota(jnp.int32, (LQ, LQ), 1) // 16
    S = jnp.where(r128 == c128, 0.5, 0.0).astype(jnp.float32)
    attn = e / jnp.dot(e, S, preferred_element_type=jnp.float32)

    lane = lax.broadcasted_iota(jnp.int32, (MA, LQ), 1)
    is_x = (lane % 2) == 0
    refsel = jnp.where(is_x, ref_ref[:, 0:1], ref_ref[:, 1:2])
    loc = jnp.clip(refsel + lo * (1.0 / HS), 0.0, 1.0)
    g = loc * HS - 0.5
    f = jnp.floor(g)
    t = g - f
    fi = f.astype(jnp.int32)
    c0 = jnp.maximum(fi, 0)
    c1 = jnp.minimum(fi + 1, HS - 1)
    wt0 = jnp.where(fi >= 0, 1.0 - t, 0.0)
    wt1 = jnp.where(fi + 1 <= HS - 1, t, 0.0)

    X = jnp.where(is_x, c0, jnp.roll(c1, 1, axis=1))
    Xw = jnp.where(is_x, wt0, jnp.roll(wt1, 1, axis=1))
    for a, (yc, yw), p_ref in (
        (0, (c0, wt0), p0_ref),
        (1, (c1, wt1), p1_ref),
    ):
        Y = jnp.where(is_x, jnp.roll(yc, -1, axis=1), yc)
        Yw = jnp.where(is_x, jnp.roll(yw, -1, axis=1), yw)
        wu = lax.bitcast_convert_type(
            (attn * Xw * Yw).astype(jnp.bfloat16), jnp.uint16)
        p_ref[...] = (Y * HS + X) | (wu.astype(jnp.int32) << 16)


def _geometry(query2d, ref2d, W_off, b_off, W_attn2, b_attn2):
    grid = (B * N // MA,)
    return pl.pallas_call(
        _geom_body,
        grid=grid,
        in_specs=[
            pl.BlockSpec((MA, D), lambda i: (i, 0)),
            pl.BlockSpec((MA, 2), lambda i: (i, 0)),
            pl.BlockSpec((D, LQ), lambda i: (0, 0)),
            pl.BlockSpec((1, LQ), lambda i: (0, 0)),
            pl.BlockSpec((D, LQ), lambda i: (0, 0)),
            pl.BlockSpec((1, LQ), lambda i: (0, 0)),
        ],
        out_specs=[
            pl.BlockSpec((MA, LQ), lambda i: (i, 0)),
            pl.BlockSpec((MA, LQ), lambda i: (i, 0)),
        ],
        out_shape=[
            jax.ShapeDtypeStruct((B * N, LQ), jnp.int32),
            jax.ShapeDtypeStruct((B * N, LQ), jnp.int32),
        ],
    )(query2d, ref2d, W_off, b_off, W_attn2, b_attn2)


MB = 512


NC = 2
NS = 16
NW = NC * NS
QPW = B * N // NW
CHUNK = 64
NCH = QPW // CHUNK
RING = 4


def _sc_body(table, p0, p1, out,
             p0_v, p1_v, idx0_v, idx1_v, w0_v, w1_v, base_v, rows, out_v,
             *sems):
    wid = lax.axis_index("s") * NC + lax.axis_index("c")
    qbase_w = wid * QPW
    b = qbase_w // N
    for hh in range(H):
        base_v[pl.ds(hh * 16, 16)] = jnp.full(
            (16,), (b * H + hh) * (HS * HS), jnp.int32)

    def copies(sl, ql):
        return (
            pltpu.make_async_copy(table.at[idx0_v.at[ql]],
                                  rows.at[sl, pl.ds(0, LQ)], sems[sl]),
            pltpu.make_async_copy(table.at[idx1_v.at[ql]],
                                  rows.at[sl, pl.ds(LQ, LQ)], sems[sl]),
        )

    def chunk_body(ci, _):
        qb = qbase_w + ci * CHUNK
        pltpu.sync_copy(p0.at[pl.ds(qb, CHUNK)], p0_v)
        pltpu.sync_copy(p1.at[pl.ds(qb, CHUNK)], p1_v)

        def prep_body(qi, _):
            for pv, iv, wv in ((p0_v, idx0_v, w0_v), (p1_v, idx1_v, w1_v)):
                for g in range(LQ // 16):
                    v = pv[qi, pl.ds(g * 16, 16)]
                    iv[qi, pl.ds(g * 16, 16)] = (
                        (v & jnp.int32(0x3FFF)) + base_v[pl.ds(g * 16, 16)])
                    wv[pl.ds(qi * LQ + g * 16, 16)] = plsc.bitcast(
                        v & jnp.int32(-65536), jnp.float32)
            return 0

        lax.fori_loop(0, CHUNK, prep_body, 0)

        for c in copies(0, 0):
            c.start()
        for c in copies(1, 1):
            c.start()

        def group_body(g, _):
            for sl in range(RING):
                ql = g * RING + sl

                @pl.when(ql + 2 < CHUNK)
                def _():
                    for c in copies((sl + 2) % RING, ql + 2):
                        c.start()

                for c in copies(sl, ql):
                    c.wait()

                def head_body(h, _):
                    prods = [[], []]
                    for a, wref in ((0, w0_v), (1, w1_v)):
                        wv16 = wref[pl.ds(ql * LQ + h * 16, 16)]
                        for l in range(16):
                            r = a * LQ + h * 16 + l
                            wf = jnp.full((16,), wv16[l], dtype=jnp.float32)
                            wb = plsc.pack(wf, wf,
                                           format=plsc.PackFormat.INTERLEAVED)
                            for j in range(2):
                                v = plsc.bitcast(
                                    rows[sl, r, pl.ds(j * 16, 16)],
                                    jnp.bfloat16)
                                prods[j].append(wb * v)
                    for j in range(2):
                        t = prods[j]
                        while len(t) > 1:
                            t = [t[i] + t[i + 1] for i in range(0, len(t), 2)]
                        out_v[ql, pl.ds(h * DH + j * 32, 32)] = t[0]
                    return 0

                lax.fori_loop(0, H, head_body, 0)
            return 0

        lax.fori_loop(0, CHUNK // RING, group_body, 0)
        pltpu.sync_copy(out_v, out.at[pl.ds(qb, CHUNK)])
        return 0

    lax.fori_loop(0, NCH, chunk_body, 0)


def _sc_gather(table, p0, p1):
    mesh = plsc.VectorSubcoreMesh(core_axis_name="c", subcore_axis_name="s")
    fn = functools.partial(
        pl.kernel,
        mesh=mesh,
        out_type=jax.ShapeDtypeStruct((B * N, D), jnp.bfloat16),
        scratch_types=[
            pltpu.VMEM((CHUNK, LQ), jnp.int32),
            pltpu.VMEM((CHUNK, LQ), jnp.int32),
            pltpu.VMEM((CHUNK, LQ), jnp.int32),
            pltpu.VMEM((CHUNK, LQ), jnp.int32),
            pltpu.VMEM((CHUNK * LQ,), jnp.float32),
            pltpu.VMEM((CHUNK * LQ,), jnp.float32),
            pltpu.VMEM((LQ,), jnp.int32),
            pltpu.VMEM((RING, 2 * LQ, DH // 2), jnp.int32),
            pltpu.VMEM((CHUNK, D), jnp.bfloat16),
        ] + [pltpu.SemaphoreType.DMA] * RING,
        compiler_params=pltpu.CompilerParams(use_tc_tiling_on_sc=False,
                                             needs_layout_passes=False),
    )(_sc_body)
    return fn(table, p0, p1)


def _wout_perm():
    perm = []
    for h in range(H):
        for j in range(2):
            for s in range(2):
                for t in range(16):
                    perm.append(h * DH + 32 * j + 2 * t + s)
    return jnp.asarray(perm, dtype=jnp.int32)


def _oproj_body(s_ref, w_ref, b_ref, out_ref):
    out_ref[...] = jnp.dot(s_ref[...], w_ref[...],
                           preferred_element_type=jnp.float32) + b_ref[...]


def _vproj_body(v_ref, w_ref, b_ref, out_ref):
    out_ref[...] = (jnp.dot(v_ref[...], w_ref[...],
                            preferred_element_type=jnp.float32)
                    + b_ref[...]).astype(jnp.bfloat16)


def _vproj(value2d, W_v, b_v):
    nb = B * N // MB
    return pl.pallas_call(
        _vproj_body,
        grid=(nb,),
        in_specs=[
            pl.BlockSpec((MB, D), lambda i: (i, 0)),
            pl.BlockSpec((D, D), lambda i: (0, 0)),
            pl.BlockSpec((1, D), lambda i: (0, 0)),
        ],
        out_specs=pl.BlockSpec((MB, D), lambda i: (i, 0)),
        out_shape=jax.ShapeDtypeStruct((B * N, D), jnp.bfloat16),
    )(value2d, W_v, b_v)


def _oproj(sampled2d, W_out, b_out):
    nb = B * N // MB
    return pl.pallas_call(
        _oproj_body,
        grid=(nb,),
        in_specs=[
            pl.BlockSpec((MB, D), lambda i: (i, 0)),
            pl.BlockSpec((D, D), lambda i: (0, 0)),
            pl.BlockSpec((1, D), lambda i: (0, 0)),
        ],
        out_specs=pl.BlockSpec((MB, D), lambda i: (i, 0)),
        out_shape=jax.ShapeDtypeStruct((B * N, D), jnp.float32),
    )(sampled2d, W_out, b_out)


def kernel(query, value, reference_points, W_off, b_off, W_attn, b_attn,
           W_v, b_v, W_out, b_out):
    query2d = query.reshape(B * N, D)
    value2d = value.reshape(B * N, D)
    ref2d = reference_points.reshape(B * N, 2)
    W_attn2 = jnp.repeat(W_attn, 2, axis=1)
    b_attn2 = jnp.repeat(b_attn, 2)[None]
    p0, p1 = _geometry(query2d, ref2d, W_off, b_off[None], W_attn2, b_attn2)
    vp = _vproj(value2d, W_v, b_v[None])
    table = lax.bitcast_convert_type(
        vp.reshape(B, N, H, DH).transpose(0, 2, 1, 3)
        .reshape(B * H * N, DH // 2, 2), jnp.int32)
    sampled = _sc_gather(table, p0, p1)
    out = _oproj(sampled, W_out.astype(jnp.bfloat16), b_out[None])
    return out.reshape(B, N, D)


# speedup vs baseline: 1.5404x; 1.5404x over previous
import functools
import math

import jax
import jax.numpy as jnp
from jax import lax
from jax.experimental import pallas as pl
from jax.experimental.pallas import tpu as pltpu
from jax.experimental.pallas import tpu_sc as plsc

B = 2
N = 16384
D = 512
H = 8
P = 8
DH = 64
HS = 128
LQ = 2 * H * P


MA = 256


def _geom_body(q_ref, ref_ref, woff_ref, boff_ref, wattn2_ref, battn2_ref,
               p0_ref, p1_ref):
    q = q_ref[...]
    lo = jnp.dot(q, woff_ref[...], preferred_element_type=jnp.float32) + boff_ref[...]
    la = jnp.dot(q, wattn2_ref[...], preferred_element_type=jnp.float32) + battn2_ref[...]
    m = jnp.max(la, axis=-1, keepdims=True)
    e = jnp.exp(la - m)
    r128 = lax.broadcasted_iota(jnp.int32, (LQ, LQ), 0) // 16
    c128 = lax.broadcasted_iota(jnp.int32, (LQ, LQ), 1) // 16
    S = jnp.where(r128 == c128, 0.5, 0.0).astype(jnp.float32)
    attn = e / jnp.dot(e, S, preferred_element_type=jnp.float32)

    lane = lax.broadcasted_iota(jnp.int32, (MA, LQ), 1)
    is_x = (lane % 2) == 0
    refsel = jnp.where(is_x, ref_ref[:, 0:1], ref_ref[:, 1:2])
    loc = jnp.clip(refsel + lo * (1.0 / HS), 0.0, 1.0)
    g = loc * HS - 0.5
    f = jnp.floor(g)
    t = g - f
    fi = f.astype(jnp.int32)
    c0 = jnp.maximum(fi, 0)
    c1 = jnp.minimum(fi + 1, HS - 1)
    wt0 = jnp.where(fi >= 0, 1.0 - t, 0.0)
    wt1 = jnp.where(fi + 1 <= HS - 1, t, 0.0)

    X = jnp.where(is_x, c0, jnp.roll(c1, 1, axis=1))
    Xw = jnp.where(is_x, wt0, jnp.roll(wt1, 1, axis=1))
    for a, (yc, yw), p_ref in (
        (0, (c0, wt0), p0_ref),
        (1, (c1, wt1), p1_ref),
    ):
        Y = jnp.where(is_x, jnp.roll(yc, -1, axis=1), yc)
        Yw = jnp.where(is_x, jnp.roll(yw, -1, axis=1), yw)
        wu = lax.bitcast_convert_type(
            (attn * Xw * Yw).astype(jnp.bfloat16), jnp.uint16)
        p_ref[...] = (Y * HS + X) | (wu.astype(jnp.int32) << 16)


def _geometry(query2d, ref2d, W_off, b_off, W_attn2, b_attn2):
    grid = (B * N // MA,)
    return pl.pallas_call(
        _geom_body,
        grid=grid,
        in_specs=[
            pl.BlockSpec((MA, D), lambda i: (i, 0)),
            pl.BlockSpec((MA, 2), lambda i: (i, 0)),
            pl.BlockSpec((D, LQ), lambda i: (0, 0)),
            pl.BlockSpec((1, LQ), lambda i: (0, 0)),
            pl.BlockSpec((D, LQ), lambda i: (0, 0)),
            pl.BlockSpec((1, LQ), lambda i: (0, 0)),
        ],
        out_specs=[
            pl.BlockSpec((MA, LQ), lambda i: (i, 0)),
            pl.BlockSpec((MA, LQ), lambda i: (i, 0)),
        ],
        out_shape=[
            jax.ShapeDtypeStruct((B * N, LQ), jnp.int32),
            jax.ShapeDtypeStruct((B * N, LQ), jnp.int32),
        ],
    )(query2d, ref2d, W_off, b_off, W_attn2, b_attn2)


MB = 512


NC = 2
NS = 16
NW = NC * NS
QPW = B * N // NW
CHUNK = 64
NCH = QPW // CHUNK
RING = 4


def _sc_body(table, p0, p1, out,
             p0_v, p1_v, idx0_v, idx1_v, w0_v, w1_v, base_v, rows, out_v,
             *sems):
    wid = lax.axis_index("s") * NC + lax.axis_index("c")
    qbase_w = wid * QPW
    b = qbase_w // N
    for hh in range(H):
        base_v[pl.ds(hh * 16, 16)] = jnp.full(
            (16,), b * N * H + hh, jnp.int32)

    def copies(sl, ql):
        return (
            pltpu.make_async_copy(table.at[idx0_v.at[ql]],
                                  rows.at[sl, pl.ds(0, LQ)], sems[sl]),
            pltpu.make_async_copy(table.at[idx1_v.at[ql]],
                                  rows.at[sl, pl.ds(LQ, LQ)], sems[sl]),
        )

    def chunk_body(ci, _):
        qb = qbase_w + ci * CHUNK
        pltpu.sync_copy(p0.at[pl.ds(qb, CHUNK)], p0_v)
        pltpu.sync_copy(p1.at[pl.ds(qb, CHUNK)], p1_v)

        def prep_body(qi, _):
            for pv, iv, wv in ((p0_v, idx0_v, w0_v), (p1_v, idx1_v, w1_v)):
                for g in range(LQ // 16):
                    v = pv[qi, pl.ds(g * 16, 16)]
                    iv[qi, pl.ds(g * 16, 16)] = (
                        ((v & jnp.int32(0x3FFF)) << 3)
                        + base_v[pl.ds(g * 16, 16)])
                    wv[pl.ds(qi * LQ + g * 16, 16)] = plsc.bitcast(
                        v & jnp.int32(-65536), jnp.float32)
            return 0

        lax.fori_loop(0, CHUNK, prep_body, 0)

        for c in copies(0, 0):
            c.start()
        for c in copies(1, 1):
            c.start()

        def group_body(g, _):
            for sl in range(RING):
                ql = g * RING + sl

                @pl.when(ql + 2 < CHUNK)
                def _():
                    for c in copies((sl + 2) % RING, ql + 2):
                        c.start()

                for c in copies(sl, ql):
                    c.wait()

                def head_body(h, _):
                    prods = [[], []]
                    for a, wref in ((0, w0_v), (1, w1_v)):
                        wv16 = wref[pl.ds(ql * LQ + h * 16, 16)]
                        for l in range(16):
                            r = a * LQ + h * 16 + l
                            wf = jnp.full((16,), wv16[l], dtype=jnp.float32)
                            wb = plsc.pack(wf, wf,
                                           format=plsc.PackFormat.INTERLEAVED)
                            for j in range(2):
                                v = plsc.bitcast(
                                    rows[sl, r, pl.ds(j * 16, 16)],
                                    jnp.bfloat16)
                                prods[j].append(wb * v)
                    for j in range(2):
                        t = prods[j]
                        while len(t) > 1:
                            t = [t[i] + t[i + 1] for i in range(0, len(t), 2)]
                        out_v[ql, pl.ds(h * DH + j * 32, 32)] = t[0]
                    return 0

                lax.fori_loop(0, H, head_body, 0)
            return 0

        lax.fori_loop(0, CHUNK // RING, group_body, 0)
        pltpu.sync_copy(out_v, out.at[pl.ds(qb, CHUNK)])
        return 0

    lax.fori_loop(0, NCH, chunk_body, 0)


def _sc_gather(table, p0, p1):
    mesh = plsc.VectorSubcoreMesh(core_axis_name="c", subcore_axis_name="s")
    fn = functools.partial(
        pl.kernel,
        mesh=mesh,
        out_type=jax.ShapeDtypeStruct((B * N, D), jnp.bfloat16),
        scratch_types=[
            pltpu.VMEM((CHUNK, LQ), jnp.int32),
            pltpu.VMEM((CHUNK, LQ), jnp.int32),
            pltpu.VMEM((CHUNK, LQ), jnp.int32),
            pltpu.VMEM((CHUNK, LQ), jnp.int32),
            pltpu.VMEM((CHUNK * LQ,), jnp.float32),
            pltpu.VMEM((CHUNK * LQ,), jnp.float32),
            pltpu.VMEM((LQ,), jnp.int32),
            pltpu.VMEM((RING, 2 * LQ, DH // 2), jnp.int32),
            pltpu.VMEM((CHUNK, D), jnp.bfloat16),
        ] + [pltpu.SemaphoreType.DMA] * RING,
        compiler_params=pltpu.CompilerParams(use_tc_tiling_on_sc=False,
                                             needs_layout_passes=False),
    )(_sc_body)
    return fn(table, p0, p1)


def _wout_perm():
    perm = []
    for h in range(H):
        for j in range(2):
            for t in range(16):
                for s in range(2):
                    perm.append(h * DH + 16 * j + t + 32 * s)
    return jnp.asarray(perm, dtype=jnp.int32)


def _oproj_body(s_ref, w_ref, b_ref, out_ref):
    out_ref[...] = jnp.dot(s_ref[...], w_ref[...],
                           preferred_element_type=jnp.float32) + b_ref[...]


def _vproj_body(v_ref, wl_ref, wh_ref, bl_ref, bh_ref, out_ref):
    v = v_ref[...]
    lo = (jnp.dot(v, wl_ref[...], preferred_element_type=jnp.float32)
          + bl_ref[...])
    hi = (jnp.dot(v, wh_ref[...], preferred_element_type=jnp.float32)
          + bh_ref[...])

    def bits16(x):
        return lax.bitcast_convert_type(
            x.astype(jnp.bfloat16), jnp.uint16).astype(jnp.int32)

    out_ref[...] = bits16(lo) | (bits16(hi) << 16)


def _vproj(value2d, W_lo, W_hi, b_lo, b_hi):
    nb = B * N // MB
    return pl.pallas_call(
        _vproj_body,
        grid=(nb,),
        in_specs=[
            pl.BlockSpec((MB, D), lambda i: (i, 0)),
            pl.BlockSpec((D, D // 2), lambda i: (0, 0)),
            pl.BlockSpec((D, D // 2), lambda i: (0, 0)),
            pl.BlockSpec((1, D // 2), lambda i: (0, 0)),
            pl.BlockSpec((1, D // 2), lambda i: (0, 0)),
        ],
        out_specs=pl.BlockSpec((MB, D // 2), lambda i: (i, 0)),
        out_shape=jax.ShapeDtypeStruct((B * N, D // 2), jnp.int32),
    )(value2d, W_lo, W_hi, b_lo, b_hi)


def _oproj(sampled2d, W_out, b_out):
    nb = B * N // MB
    return pl.pallas_call(
        _oproj_body,
        grid=(nb,),
        in_specs=[
            pl.BlockSpec((MB, D), lambda i: (i, 0)),
            pl.BlockSpec((D, D), lambda i: (0, 0)),
            pl.BlockSpec((1, D), lambda i: (0, 0)),
        ],
        out_specs=pl.BlockSpec((MB, D), lambda i: (i, 0)),
        out_shape=jax.ShapeDtypeStruct((B * N, D), jnp.float32),
    )(sampled2d, W_out, b_out)


def kernel(query, value, reference_points, W_off, b_off, W_attn, b_attn,
           W_v, b_v, W_out, b_out):
    query2d = query.reshape(B * N, D)
    value2d = value.reshape(B * N, D)
    ref2d = reference_points.reshape(B * N, 2)
    W_attn2 = jnp.repeat(W_attn, 2, axis=1)
    b_attn2 = jnp.repeat(b_attn, 2)[None]
    p0, p1 = _geometry(query2d, ref2d, W_off, b_off[None], W_attn2, b_attn2)
    mcols = jnp.arange(D // 2, dtype=jnp.int32)
    lo_cols = (mcols // 32) * DH + mcols % 32
    hi_cols = lo_cols + 32
    table = _vproj(value2d, W_v[:, lo_cols], W_v[:, hi_cols],
                   b_v[lo_cols][None], b_v[hi_cols][None])
    table = table.reshape(B * N * H, DH // 2)
    sampled = _sc_gather(table, p0, p1)
    out = _oproj(sampled, W_out.astype(jnp.bfloat16)[_wout_perm()],
                 b_out[None])
    return out.reshape(B, N, D)


# speedup vs baseline: 1.7220x; 1.1178x over previous
import functools
import math

import jax
import jax.numpy as jnp
from jax import lax
from jax.experimental import pallas as pl
from jax.experimental.pallas import tpu as pltpu
from jax.experimental.pallas import tpu_sc as plsc

B = 2
N = 16384
D = 512
H = 8
P = 8
DH = 64
HS = 128
LQ = 2 * H * P


MA = 512


def _geom_body(q_ref, ref_ref, woff_ref, boff_ref, wattn2_ref, battn2_ref,
               p0_ref, p1_ref):
    q = q_ref[...]
    lo = jnp.dot(q, woff_ref[...], preferred_element_type=jnp.float32) + boff_ref[...]
    la = jnp.dot(q, wattn2_ref[...], preferred_element_type=jnp.float32) + battn2_ref[...]
    m = jnp.max(la, axis=-1, keepdims=True)
    e = jnp.exp(la - m)
    r128 = lax.broadcasted_iota(jnp.int32, (LQ, LQ), 0) // 16
    c128 = lax.broadcasted_iota(jnp.int32, (LQ, LQ), 1) // 16
    S = jnp.where(r128 == c128, 0.5, 0.0).astype(jnp.float32)
    attn = e / jnp.dot(e, S, preferred_element_type=jnp.float32)

    lane = lax.broadcasted_iota(jnp.int32, (MA, LQ), 1)
    is_x = (lane % 2) == 0
    refsel = jnp.where(is_x, ref_ref[:, 0:1], ref_ref[:, 1:2])
    loc = jnp.clip(refsel + lo * (1.0 / HS), 0.0, 1.0)
    g = loc * HS - 0.5
    f = jnp.floor(g)
    t = g - f
    fi = f.astype(jnp.int32)
    c0 = jnp.maximum(fi, 0)
    c1 = jnp.minimum(fi + 1, HS - 1)
    wt0 = jnp.where(fi >= 0, 1.0 - t, 0.0)
    wt1 = jnp.where(fi + 1 <= HS - 1, t, 0.0)

    X = jnp.where(is_x, c0, jnp.roll(c1, 1, axis=1))
    Xw = jnp.where(is_x, wt0, jnp.roll(wt1, 1, axis=1))
    for a, (yc, yw), p_ref in (
        (0, (c0, wt0), p0_ref),
        (1, (c1, wt1), p1_ref),
    ):
        Y = jnp.where(is_x, jnp.roll(yc, -1, axis=1), yc)
        Yw = jnp.where(is_x, jnp.roll(yw, -1, axis=1), yw)
        wu = lax.bitcast_convert_type(
            (attn * Xw * Yw).astype(jnp.bfloat16), jnp.uint16)
        p_ref[...] = (Y * HS + X) | (wu.astype(jnp.int32) << 16)


def _geometry(query2d, ref2d, W_off, b_off, W_attn2, b_attn2):
    grid = (B * N // MA,)
    return pl.pallas_call(
        _geom_body,
        grid=grid,
        in_specs=[
            pl.BlockSpec((MA, D), lambda i: (i, 0)),
            pl.BlockSpec((MA, 2), lambda i: (i, 0)),
            pl.BlockSpec((D, LQ), lambda i: (0, 0)),
            pl.BlockSpec((1, LQ), lambda i: (0, 0)),
            pl.BlockSpec((D, LQ), lambda i: (0, 0)),
            pl.BlockSpec((1, LQ), lambda i: (0, 0)),
        ],
        out_specs=[
            pl.BlockSpec((MA, LQ), lambda i: (i, 0)),
            pl.BlockSpec((MA, LQ), lambda i: (i, 0)),
        ],
        out_shape=[
            jax.ShapeDtypeStruct((B * N, LQ), jnp.int32),
            jax.ShapeDtypeStruct((B * N, LQ), jnp.int32),
        ],
    )(query2d, ref2d, W_off, b_off, W_attn2, b_attn2)


MB = 512


NC = 2
NS = 16
NW = NC * NS
QPW = B * N // NW
CHUNK = 64
NCH = QPW // CHUNK
RING = 4


def _sc_body(table, p0, p1, out,
             p0_v, p1_v, idx0_v, idx1_v, w0_v, w1_v, base_v, rows, out_v,
             *sems):
    wid = lax.axis_index("s") * NC + lax.axis_index("c")
    qbase_w = wid * QPW
    b = qbase_w // N
    for hh in range(H):
        base_v[pl.ds(hh * 16, 16)] = jnp.full(
            (16,), b * N * H + hh, jnp.int32)

    def copies(sl, ql):
        return (
            pltpu.make_async_copy(table.at[idx0_v.at[ql]],
                                  rows.at[sl, pl.ds(0, LQ)], sems[sl]),
            pltpu.make_async_copy(table.at[idx1_v.at[ql]],
                                  rows.at[sl, pl.ds(LQ, LQ)], sems[sl]),
        )

    def chunk_body(ci, _):
        qb = qbase_w + ci * CHUNK
        pltpu.sync_copy(p0.at[pl.ds(qb, CHUNK)], p0_v)
        pltpu.sync_copy(p1.at[pl.ds(qb, CHUNK)], p1_v)

        def prep_body(qi, _):
            for pv, iv, wv in ((p0_v, idx0_v, w0_v), (p1_v, idx1_v, w1_v)):
                for g in range(LQ // 16):
                    v = pv[qi, pl.ds(g * 16, 16)]
                    iv[qi, pl.ds(g * 16, 16)] = (
                        ((v & jnp.int32(0x3FFF)) << 3)
                        + base_v[pl.ds(g * 16, 16)])
                    wv[pl.ds(qi * LQ + g * 16, 16)] = plsc.bitcast(
                        v & jnp.int32(-65536), jnp.float32)
            return 0

        lax.fori_loop(0, CHUNK, prep_body, 0)

        for sl in range(3):
            for c in copies(sl, sl):
                c.start()

        def group_body(g, _):
            for sl in range(RING):
                ql = g * RING + sl

                @pl.when(ql + 3 < CHUNK)
                def _():
                    for c in copies((sl + 3) % RING, ql + 3):
                        c.start()

                for c in copies(sl, ql):
                    c.wait()

                def head_body(h, _):
                    prods = [[], []]
                    for a, wref in ((0, w0_v), (1, w1_v)):
                        wv16 = wref[pl.ds(ql * LQ + h * 16, 16)]
                        for l in range(16):
                            r = a * LQ + h * 16 + l
                            wf = jnp.full((16,), wv16[l], dtype=jnp.float32)
                            wb = plsc.pack(wf, wf,
                                           format=plsc.PackFormat.INTERLEAVED)
                            for j in range(2):
                                v = plsc.bitcast(
                                    rows[sl, r, pl.ds(j * 16, 16)],
                                    jnp.bfloat16)
                                prods[j].append(wb * v)
                    for j in range(2):
                        t = prods[j]
                        while len(t) > 1:
                            t = [t[i] + t[i + 1] for i in range(0, len(t), 2)]
                        out_v[ql, pl.ds(h * DH + j * 32, 32)] = t[0]
                    return 0

                lax.fori_loop(0, H, head_body, 0)
            return 0

        lax.fori_loop(0, CHUNK // RING, group_body, 0)
        pltpu.sync_copy(out_v, out.at[pl.ds(qb, CHUNK)])
        return 0

    lax.fori_loop(0, NCH, chunk_body, 0)


def _sc_gather(table, p0, p1):
    mesh = plsc.VectorSubcoreMesh(core_axis_name="c", subcore_axis_name="s")
    fn = functools.partial(
        pl.kernel,
        mesh=mesh,
        out_type=jax.ShapeDtypeStruct((B * N, D), jnp.bfloat16),
        scratch_types=[
            pltpu.VMEM((CHUNK, LQ), jnp.int32),
            pltpu.VMEM((CHUNK, LQ), jnp.int32),
            pltpu.VMEM((CHUNK, LQ), jnp.int32),
            pltpu.VMEM((CHUNK, LQ), jnp.int32),
            pltpu.VMEM((CHUNK * LQ,), jnp.float32),
            pltpu.VMEM((CHUNK * LQ,), jnp.float32),
            pltpu.VMEM((LQ,), jnp.int32),
            pltpu.VMEM((RING, 2 * LQ, DH // 2), jnp.int32),
            pltpu.VMEM((CHUNK, D), jnp.bfloat16),
        ] + [pltpu.SemaphoreType.DMA] * RING,
        compiler_params=pltpu.CompilerParams(use_tc_tiling_on_sc=False,
                                             needs_layout_passes=False),
    )(_sc_body)
    return fn(table, p0, p1)


def _wout_perm():
    perm = []
    for h in range(H):
        for j in range(2):
            for t in range(16):
                for s in range(2):
                    perm.append(h * DH + 16 * j + t + 32 * s)
    return jnp.asarray(perm, dtype=jnp.int32)


def _oproj_body(s_ref, w_ref, b_ref, out_ref):
    out_ref[...] = jnp.dot(s_ref[...], w_ref[...],
                           preferred_element_type=jnp.float32) + b_ref[...]


def _vproj_body(v_ref, wl_ref, wh_ref, bl_ref, bh_ref, out_ref):
    v = v_ref[...]
    lo = (jnp.dot(v, wl_ref[...], preferred_element_type=jnp.float32)
          + bl_ref[...])
    hi = (jnp.dot(v, wh_ref[...], preferred_element_type=jnp.float32)
          + bh_ref[...])

    def bits16(x):
        return lax.bitcast_convert_type(
            x.astype(jnp.bfloat16), jnp.uint16).astype(jnp.int32)

    out_ref[...] = bits16(lo) | (bits16(hi) << 16)


def _vproj(value2d, W_lo, W_hi, b_lo, b_hi):
    nb = B * N // MB
    return pl.pallas_call(
        _vproj_body,
        grid=(nb,),
        in_specs=[
            pl.BlockSpec((MB, D), lambda i: (i, 0)),
            pl.BlockSpec((D, D // 2), lambda i: (0, 0)),
            pl.BlockSpec((D, D // 2), lambda i: (0, 0)),
            pl.BlockSpec((1, D // 2), lambda i: (0, 0)),
            pl.BlockSpec((1, D // 2), lambda i: (0, 0)),
        ],
        out_specs=pl.BlockSpec((MB, D // 2), lambda i: (i, 0)),
        out_shape=jax.ShapeDtypeStruct((B * N, D // 2), jnp.int32),
    )(value2d, W_lo, W_hi, b_lo, b_hi)


def _oproj(sampled2d, W_out, b_out):
    nb = B * N // MB
    return pl.pallas_call(
        _oproj_body,
        grid=(nb,),
        in_specs=[
            pl.BlockSpec((MB, D), lambda i: (i, 0)),
            pl.BlockSpec((D, D), lambda i: (0, 0)),
            pl.BlockSpec((1, D), lambda i: (0, 0)),
        ],
        out_specs=pl.BlockSpec((MB, D), lambda i: (i, 0)),
        out_shape=jax.ShapeDtypeStruct((B * N, D), jnp.float32),
    )(sampled2d, W_out, b_out)


def kernel(query, value, reference_points, W_off, b_off, W_attn, b_attn,
           W_v, b_v, W_out, b_out):
    query2d = query.reshape(B * N, D)
    value2d = value.reshape(B * N, D)
    ref2d = reference_points.reshape(B * N, 2)
    W_attn2 = jnp.repeat(W_attn, 2, axis=1)
    b_attn2 = jnp.repeat(b_attn, 2)[None]
    p0, p1 = _geometry(query2d, ref2d, W_off, b_off[None], W_attn2, b_attn2)
    mcols = jnp.arange(D // 2, dtype=jnp.int32)
    lo_cols = (mcols // 32) * DH + mcols % 32
    hi_cols = lo_cols + 32
    table = _vproj(value2d, W_v[:, lo_cols], W_v[:, hi_cols],
                   b_v[lo_cols][None], b_v[hi_cols][None])
    table = table.reshape(B * N * H, DH // 2)
    sampled = _sc_gather(table, p0, p1)
    out = _oproj(sampled, W_out.astype(jnp.bfloat16)[_wout_perm()],
                 b_out[None])
    return out.reshape(B, N, D)
